# baseline (speedup 1.0000x reference)
import functools

import jax
import jax.numpy as jnp
import numpy as np
from jax import lax
from jax.experimental import pallas as pl
from jax.experimental.pallas import tpu as pltpu

N_BINS = 15


def _ece_body(nrows, grid, logits_ref, labels_ref, out_ref, acc_ref):
    i = pl.program_id(0)

    @pl.when(i == 0)
    def _init():
        acc_ref[...] = jnp.zeros_like(acc_ref)

    l = logits_ref[...]
    lab = labels_ref[0]
    b, c = l.shape

    m = jnp.max(l, axis=1, keepdims=True)
    z = jnp.sum(jnp.exp(l - m), axis=1, keepdims=True)
    conf = 1.0 / z

    iota = lax.broadcasted_iota(jnp.int32, (b, c), 1)
    pred = jnp.min(jnp.where(l == m, iota, c), axis=1, keepdims=True)
    acc = (pred == lab).astype(jnp.float32)

    bidx = lax.broadcasted_iota(jnp.int32, (1, N_BINS), 1).astype(jnp.float32)
    lows = bidx / N_BINS
    highs = (bidx + 1.0) / N_BINS
    in_bin = ((conf > lows) & (conf <= highs)).astype(jnp.float32)

    cnt = jnp.sum(in_bin, axis=0, keepdims=True)
    asum = jnp.sum(in_bin * acc, axis=0, keepdims=True)
    csum = jnp.sum(in_bin * conf, axis=0, keepdims=True)

    acc_ref[0:1, 0:N_BINS] += cnt
    acc_ref[1:2, 0:N_BINS] += asum
    acc_ref[2:3, 0:N_BINS] += csum

    @pl.when(i == grid - 1)
    def _combine():
        tot_cnt = acc_ref[0:1, 0:N_BINS]
        tot_asum = acc_ref[1:2, 0:N_BINS]
        tot_csum = acc_ref[2:3, 0:N_BINS]
        prop = tot_cnt / nrows
        safe = jnp.maximum(tot_cnt, 1.0)
        acc_mean = tot_asum / safe
        conf_mean = tot_csum / safe
        nonempty = (tot_cnt > 0).astype(jnp.float32)
        per_bin = jnp.abs(conf_mean - acc_mean) * prop * nonempty
        out_ref[...] = jnp.sum(per_bin, axis=1, keepdims=True)


def kernel(logits, labels):
    n, c = logits.shape
    block = 8000
    assert n % block == 0
    grid = n // block
    labels2 = labels.astype(jnp.int32).reshape(grid, block, 1)

    out = pl.pallas_call(
        functools.partial(_ece_body, float(n), grid),
        grid=(grid,),
        in_specs=[
            pl.BlockSpec((block, c), lambda i: (i, 0)),
            pl.BlockSpec((1, block, 1), lambda i: (i, 0, 0)),
        ],
        out_specs=pl.BlockSpec((1, 1), lambda i: (0, 0)),
        out_shape=jax.ShapeDtypeStruct((1, 1), jnp.float32),
        scratch_shapes=[pltpu.VMEM((8, 128), jnp.float32)],
    )(logits, labels2)
    return out.reshape(1)


# speedup vs baseline: 1.0556x; 1.0556x over previous
import functools

import jax
import jax.numpy as jnp
from jax import lax
from jax.experimental import pallas as pl
from jax.experimental.pallas import tpu as pltpu
from jax.experimental.pallas import tpu_sc as plsc

N_BINS = 15
N_ROWS = 1_000_000
N_CLS = 64
NW = 32
ROWS_W = 31_256
ROWS_LAST = N_ROWS - 31 * ROWS_W
CHUNK = 624
FULL_CHUNKS = ROWS_W // CHUNK
TAIL = ROWS_W - FULL_CHUNKS * CHUNK
FULL_CHUNKS_LAST = ROWS_LAST // CHUNK
TAIL_LAST = ROWS_LAST - FULL_CHUNKS_LAST * CHUNK


def _iota16():
    return lax.iota(jnp.int32, 16)


def _process_group(lbuf, labbuf, hists, row_local, n_valid):
    cnt_h, acc_h, conf_h = hists
    lanes = _iota16()
    idx0 = (row_local + lanes) * N_CLS
    m = jnp.full((16,), -jnp.inf, dtype=jnp.float32)
    s = jnp.zeros((16,), dtype=jnp.float32)
    for j in range(N_CLS):
        v = plsc.load_gather(lbuf, [idx0 + j])
        s = s + jnp.exp(v)
        m = jnp.maximum(m, v)
    conf = jnp.exp(m) / s

    lab = plsc.load_gather(labbuf, [row_local + lanes])
    if n_valid < 16:
        lab = jnp.clip(lab, 0, N_CLS - 1)
    v_lab = plsc.load_gather(lbuf, [idx0 + lab])
    acc = jnp.where(v_lab == m, 1.0, 0.0).astype(jnp.float32)

    binf = conf * jnp.float32(N_BINS)
    bi = binf.astype(jnp.int32)
    exact = (bi.astype(jnp.float32) == binf).astype(jnp.int32)
    bi = jnp.minimum(bi - exact, N_BINS - 1)

    slots = lanes * 16 + bi
    ones = jnp.ones((16,), dtype=jnp.float32)
    if n_valid < 16:
        msk = lanes < n_valid
        plsc.addupdate_scatter(cnt_h, [slots], ones, mask=msk)
        plsc.addupdate_scatter(acc_h, [slots], acc, mask=msk)
        plsc.addupdate_scatter(conf_h, [slots], conf, mask=msk)
    else:
        plsc.addupdate_scatter(cnt_h, [slots], ones)
        plsc.addupdate_scatter(acc_h, [slots], acc)
        plsc.addupdate_scatter(conf_h, [slots], conf)


def _sc_partials(logits_flat, labels):
    mesh = plsc.VectorSubcoreMesh(core_axis_name="c", subcore_axis_name="s")

    @functools.partial(
        pl.kernel,
        mesh=mesh,
        out_type=jax.ShapeDtypeStruct((NW, 48), jnp.float32),
        compiler_params=pltpu.CompilerParams(needs_layout_passes=False),
        scratch_types=[
            pltpu.VMEM((CHUNK * N_CLS,), jnp.float32),
            pltpu.VMEM((CHUNK,), jnp.int32),
            pltpu.VMEM((256,), jnp.float32),
            pltpu.VMEM((256,), jnp.float32),
            pltpu.VMEM((256,), jnp.float32),
            pltpu.VMEM((48,), jnp.float32),
        ],
    )
    def body(logits_hbm, labels_hbm, out_hbm, lbuf, labbuf,
             cnt_h, acc_h, conf_h, stage):
        wid = lax.axis_index("s") * 2 + lax.axis_index("c")
        row0 = wid * ROWS_W
        lanes = _iota16()
        z16 = jnp.zeros((16,), dtype=jnp.float32)
        for k in range(16):
            cnt_h[pl.ds(k * 16, 16)] = z16
            acc_h[pl.ds(k * 16, 16)] = z16
            conf_h[pl.ds(k * 16, 16)] = z16
        hists = (cnt_h, acc_h, conf_h)

        def do_chunk(base, nrows, ngroups, tail_valid):
            pltpu.sync_copy(
                logits_hbm.at[pl.ds(base * N_CLS, nrows * N_CLS)],
                lbuf.at[pl.ds(0, nrows * N_CLS)])
            pltpu.sync_copy(labels_hbm.at[pl.ds(base, nrows)],
                            labbuf.at[pl.ds(0, nrows)])

            def g_body(g, carry):
                _process_group(lbuf, labbuf, hists, g * 16, 16)
                return carry
            lax.fori_loop(0, ngroups, g_body, 0)
            if tail_valid:
                _process_group(lbuf, labbuf, hists, ngroups * 16, tail_valid)

        nfull = jnp.where(wid == NW - 1, FULL_CHUNKS_LAST, FULL_CHUNKS)

        def c_body(c, carry):
            do_chunk(row0 + c * CHUNK, CHUNK, CHUNK // 16, 0)
            return carry
        lax.fori_loop(0, nfull, c_body, 0)

        @pl.when(wid < NW - 1)
        def _tail_main():
            do_chunk(row0 + FULL_CHUNKS * CHUNK, TAIL, TAIL // 16, 8)

        @pl.when(wid == NW - 1)
        def _tail_last():
            do_chunk(row0 + FULL_CHUNKS_LAST * CHUNK, TAIL_LAST,
                     TAIL_LAST // 16, 8)

        for h_idx, h in enumerate(hists):
            tot = z16
            for lane in range(16):
                tot = tot + plsc.load_gather(h, [lane * 16 + lanes])
            stage[pl.ds(h_idx * 16, 16)] = tot
        pltpu.sync_copy(stage, out_hbm.at[wid])

    return body(logits_flat, labels)


def _combine_body(nrows, p_ref, out_ref):
    p = p_ref[...]
    cnt = jnp.sum(p[:, 0:N_BINS], axis=0, keepdims=True)
    asum = jnp.sum(p[:, 16:16 + N_BINS], axis=0, keepdims=True)
    csum = jnp.sum(p[:, 32:32 + N_BINS], axis=0, keepdims=True)
    prop = cnt / nrows
    safe = jnp.maximum(cnt, 1.0)
    nonempty = (cnt > 0).astype(jnp.float32)
    per_bin = jnp.abs(csum / safe - asum / safe) * prop * nonempty
    out_ref[...] = jnp.sum(per_bin, axis=1, keepdims=True)


def kernel(logits, labels):
    n, c = logits.shape
    partials = _sc_partials(logits.reshape(n * c), labels.astype(jnp.int32))
    out = pl.pallas_call(
        functools.partial(_combine_body, float(n)),
        out_shape=jax.ShapeDtypeStruct((1, 1), jnp.float32),
    )(partials)
    return out.reshape(1)


# speedup vs baseline: 1.6132x; 1.5282x over previous
import functools

import jax
import jax.numpy as jnp
from jax import lax
from jax.experimental import pallas as pl
from jax.experimental.pallas import tpu as pltpu
from jax.experimental.pallas import tpu_sc as plsc

N_BINS = 15
N_ROWS = 1_000_000
N_CLS = 64
NW = 32
ROWS_W = 31_256
ROWS_LAST = N_ROWS - 31 * ROWS_W
CHUNK = 624
FULL_CHUNKS = ROWS_W // CHUNK
TAIL = ROWS_W - FULL_CHUNKS * CHUNK
FULL_CHUNKS_LAST = ROWS_LAST // CHUNK
TAIL_LAST = ROWS_LAST - FULL_CHUNKS_LAST * CHUNK


def _iota16():
    return lax.iota(jnp.int32, 16)


def _process_group(lbuf, labbuf, hists, row_local, n_valid):
    cnt_h, acc_h, conf_h = hists
    lanes = _iota16()
    idx0 = (row_local + lanes) * N_CLS
    m = jnp.full((16,), -jnp.inf, dtype=jnp.float32)
    s = jnp.zeros((16,), dtype=jnp.float32)
    for j in range(N_CLS):
        col = (lanes + j) & (N_CLS - 1)
        v = plsc.load_gather(lbuf, [idx0 + col])
        s = s + jnp.exp(v)
        m = jnp.maximum(m, v)
    conf = jnp.exp(m) / s

    lab = plsc.load_gather(labbuf, [row_local + lanes])
    if n_valid < 16:
        lab = jnp.clip(lab, 0, N_CLS - 1)
    v_lab = plsc.load_gather(lbuf, [idx0 + lab])
    acc = jnp.where(v_lab == m, 1.0, 0.0).astype(jnp.float32)

    binf = conf * jnp.float32(N_BINS)
    bi = binf.astype(jnp.int32)
    exact = (bi.astype(jnp.float32) == binf).astype(jnp.int32)
    bi = jnp.minimum(bi - exact, N_BINS - 1)

    slots = lanes * 16 + bi
    ones = jnp.ones((16,), dtype=jnp.float32)
    if n_valid < 16:
        msk = lanes < n_valid
        plsc.addupdate_scatter(cnt_h, [slots], ones, mask=msk)
        plsc.addupdate_scatter(acc_h, [slots], acc, mask=msk)
        plsc.addupdate_scatter(conf_h, [slots], conf, mask=msk)
    else:
        plsc.addupdate_scatter(cnt_h, [slots], ones)
        plsc.addupdate_scatter(acc_h, [slots], acc)
        plsc.addupdate_scatter(conf_h, [slots], conf)


def _sc_partials(logits_flat, labels):
    mesh = plsc.VectorSubcoreMesh(core_axis_name="c", subcore_axis_name="s")

    @functools.partial(
        pl.kernel,
        mesh=mesh,
        out_type=jax.ShapeDtypeStruct((NW, 48), jnp.float32),
        compiler_params=pltpu.CompilerParams(needs_layout_passes=False),
        scratch_types=[
            pltpu.VMEM((CHUNK * N_CLS,), jnp.float32),
            pltpu.VMEM((CHUNK,), jnp.int32),
            pltpu.VMEM((256,), jnp.float32),
            pltpu.VMEM((256,), jnp.float32),
            pltpu.VMEM((256,), jnp.float32),
            pltpu.VMEM((48,), jnp.float32),
        ],
    )
    def body(logits_hbm, labels_hbm, out_hbm, lbuf, labbuf,
             cnt_h, acc_h, conf_h, stage):
        wid = lax.axis_index("s") * 2 + lax.axis_index("c")
        row0 = wid * ROWS_W
        lanes = _iota16()
        z16 = jnp.zeros((16,), dtype=jnp.float32)
        for k in range(16):
            cnt_h[pl.ds(k * 16, 16)] = z16
            acc_h[pl.ds(k * 16, 16)] = z16
            conf_h[pl.ds(k * 16, 16)] = z16
        hists = (cnt_h, acc_h, conf_h)

        def do_chunk(base, nrows, ngroups, tail_valid):
            pltpu.sync_copy(
                logits_hbm.at[pl.ds(base * N_CLS, nrows * N_CLS)],
                lbuf.at[pl.ds(0, nrows * N_CLS)])
            pltpu.sync_copy(labels_hbm.at[pl.ds(base, nrows)],
                            labbuf.at[pl.ds(0, nrows)])

            def g_body(g, carry):
                _process_group(lbuf, labbuf, hists, g * 16, 16)
                return carry
            lax.fori_loop(0, ngroups, g_body, 0)
            if tail_valid:
                _process_group(lbuf, labbuf, hists, ngroups * 16, tail_valid)

        nfull = jnp.where(wid == NW - 1, FULL_CHUNKS_LAST, FULL_CHUNKS)

        def c_body(c, carry):
            do_chunk(row0 + c * CHUNK, CHUNK, CHUNK // 16, 0)
            return carry
        lax.fori_loop(0, nfull, c_body, 0)

        @pl.when(wid < NW - 1)
        def _tail_main():
            do_chunk(row0 + FULL_CHUNKS * CHUNK, TAIL, TAIL // 16, 8)

        @pl.when(wid == NW - 1)
        def _tail_last():
            do_chunk(row0 + FULL_CHUNKS_LAST * CHUNK, TAIL_LAST,
                     TAIL_LAST // 16, 8)

        for h_idx, h in enumerate(hists):
            tot = z16
            for lane in range(16):
                tot = tot + plsc.load_gather(h, [lane * 16 + lanes])
            stage[pl.ds(h_idx * 16, 16)] = tot
        pltpu.sync_copy(stage, out_hbm.at[wid])

    return body(logits_flat, labels)


def _combine_body(nrows, p_ref, out_ref):
    p = p_ref[...]
    cnt = jnp.sum(p[:, 0:N_BINS], axis=0, keepdims=True)
    asum = jnp.sum(p[:, 16:16 + N_BINS], axis=0, keepdims=True)
    csum = jnp.sum(p[:, 32:32 + N_BINS], axis=0, keepdims=True)
    prop = cnt / nrows
    safe = jnp.maximum(cnt, 1.0)
    nonempty = (cnt > 0).astype(jnp.float32)
    per_bin = jnp.abs(csum / safe - asum / safe) * prop * nonempty
    out_ref[...] = jnp.sum(per_bin, axis=1, keepdims=True)


def kernel(logits, labels):
    n, c = logits.shape
    partials = _sc_partials(logits.reshape(n * c), labels.astype(jnp.int32))
    out = pl.pallas_call(
        functools.partial(_combine_body, float(n)),
        out_shape=jax.ShapeDtypeStruct((1, 1), jnp.float32),
    )(partials)
    return out.reshape(1)


# speedup vs baseline: 2.0839x; 1.2918x over previous
import functools

import jax
import jax.numpy as jnp
from jax import lax
from jax.experimental import pallas as pl
from jax.experimental.pallas import tpu as pltpu
from jax.experimental.pallas import tpu_sc as plsc

N_BINS = 15
N_ROWS = 1_000_000
N_CLS = 64
NW = 32
ROWS_W = 31_256
ROWS_LAST = N_ROWS - 31 * ROWS_W
CHUNK = 624
FULL_CHUNKS = ROWS_W // CHUNK
TAIL = ROWS_W - FULL_CHUNKS * CHUNK
FULL_CHUNKS_LAST = ROWS_LAST // CHUNK
TAIL_LAST = ROWS_LAST - FULL_CHUNKS_LAST * CHUNK


def _iota16():
    return lax.iota(jnp.int32, 16)


def _process_group(lbuf, labbuf, hists, row_local, n_valid):
    cnt_h, acc_h, conf_h = hists
    lanes = _iota16()
    rows = row_local + lanes
    ms = [jnp.full((16,), -jnp.inf, dtype=jnp.float32) for _ in range(4)]
    ss = [jnp.zeros((16,), dtype=jnp.float32) for _ in range(4)]
    for j in range(N_CLS):
        col = (lanes + j) & (N_CLS - 1)
        v = plsc.load_gather(lbuf, [rows, col])
        ss[j % 4] = ss[j % 4] + jnp.exp(v)
        ms[j % 4] = jnp.maximum(ms[j % 4], v)
    s = (ss[0] + ss[1]) + (ss[2] + ss[3])
    m = jnp.maximum(jnp.maximum(ms[0], ms[1]), jnp.maximum(ms[2], ms[3]))
    conf = jnp.exp(m) / s

    lab = plsc.load_gather(labbuf, [rows])
    if n_valid < 16:
        lab = jnp.clip(lab, 0, N_CLS - 1)
    v_lab = plsc.load_gather(lbuf, [rows, lab])
    acc = jnp.where(v_lab == m, 1.0, 0.0).astype(jnp.float32)

    binf = conf * jnp.float32(N_BINS)
    bi = binf.astype(jnp.int32)
    exact = (bi.astype(jnp.float32) == binf).astype(jnp.int32)
    bi = jnp.minimum(bi - exact, N_BINS - 1)

    slots = lanes * 16 + bi
    ones = jnp.ones((16,), dtype=jnp.float32)
    if n_valid < 16:
        msk = lanes < n_valid
        plsc.addupdate_scatter(cnt_h, [slots], ones, mask=msk)
        plsc.addupdate_scatter(acc_h, [slots], acc, mask=msk)
        plsc.addupdate_scatter(conf_h, [slots], conf, mask=msk)
    else:
        plsc.addupdate_scatter(cnt_h, [slots], ones)
        plsc.addupdate_scatter(acc_h, [slots], acc)
        plsc.addupdate_scatter(conf_h, [slots], conf)


def _sc_partials(logits_flat, labels):
    mesh = plsc.VectorSubcoreMesh(core_axis_name="c", subcore_axis_name="s")

    @functools.partial(
        pl.kernel,
        mesh=mesh,
        out_type=jax.ShapeDtypeStruct((NW, 48), jnp.float32),
        compiler_params=pltpu.CompilerParams(needs_layout_passes=False),
        scratch_types=[
            pltpu.VMEM((CHUNK, N_CLS), jnp.float32),
            pltpu.VMEM((CHUNK,), jnp.int32),
            pltpu.VMEM((256,), jnp.float32),
            pltpu.VMEM((256,), jnp.float32),
            pltpu.VMEM((256,), jnp.float32),
            pltpu.VMEM((48,), jnp.float32),
        ],
    )
    def body(logits_hbm, labels_hbm, out_hbm, lbuf, labbuf,
             cnt_h, acc_h, conf_h, stage):
        wid = lax.axis_index("s") * 2 + lax.axis_index("c")
        row0 = wid * ROWS_W
        lanes = _iota16()
        z16 = jnp.zeros((16,), dtype=jnp.float32)
        for k in range(16):
            cnt_h[pl.ds(k * 16, 16)] = z16
            acc_h[pl.ds(k * 16, 16)] = z16
            conf_h[pl.ds(k * 16, 16)] = z16
        hists = (cnt_h, acc_h, conf_h)

        def do_chunk(base, nrows, ngroups, tail_valid):
            pltpu.sync_copy(
                logits_hbm.at[pl.ds(base, nrows)],
                lbuf.at[pl.ds(0, nrows)])
            pltpu.sync_copy(labels_hbm.at[pl.ds(base, nrows)],
                            labbuf.at[pl.ds(0, nrows)])

            def g_body(g, carry):
                _process_group(lbuf, labbuf, hists, g * 16, 16)
                return carry
            lax.fori_loop(0, ngroups, g_body, 0)
            if tail_valid:
                _process_group(lbuf, labbuf, hists, ngroups * 16, tail_valid)

        nfull = jnp.where(wid == NW - 1, FULL_CHUNKS_LAST, FULL_CHUNKS)

        def c_body(c, carry):
            do_chunk(row0 + c * CHUNK, CHUNK, CHUNK // 16, 0)
            return carry
        lax.fori_loop(0, nfull, c_body, 0)

        @pl.when(wid < NW - 1)
        def _tail_main():
            do_chunk(row0 + FULL_CHUNKS * CHUNK, TAIL, TAIL // 16, 8)

        @pl.when(wid == NW - 1)
        def _tail_last():
            do_chunk(row0 + FULL_CHUNKS_LAST * CHUNK, TAIL_LAST,
                     TAIL_LAST // 16, 8)

        for h_idx, h in enumerate(hists):
            tot = z16
            for lane in range(16):
                tot = tot + plsc.load_gather(h, [lane * 16 + lanes])
            stage[pl.ds(h_idx * 16, 16)] = tot
        pltpu.sync_copy(stage, out_hbm.at[wid])

    return body(logits_flat, labels)


def _combine_body(nrows, p_ref, out_ref):
    p = p_ref[...]
    cnt = jnp.sum(p[:, 0:N_BINS], axis=0, keepdims=True)
    asum = jnp.sum(p[:, 16:16 + N_BINS], axis=0, keepdims=True)
    csum = jnp.sum(p[:, 32:32 + N_BINS], axis=0, keepdims=True)
    prop = cnt / nrows
    safe = jnp.maximum(cnt, 1.0)
    nonempty = (cnt > 0).astype(jnp.float32)
    per_bin = jnp.abs(csum / safe - asum / safe) * prop * nonempty
    out_ref[...] = jnp.sum(per_bin, axis=1, keepdims=True)


def kernel(logits, labels):
    n, c = logits.shape
    partials = _sc_partials(logits, labels.astype(jnp.int32))
    out = pl.pallas_call(
        functools.partial(_combine_body, float(n)),
        out_shape=jax.ShapeDtypeStruct((1, 1), jnp.float32),
    )(partials)
    return out.reshape(1)


# speedup vs baseline: 2.6318x; 1.2630x over previous
import functools

import jax
import jax.numpy as jnp
from jax import lax
from jax.experimental import pallas as pl
from jax.experimental.pallas import tpu as pltpu
from jax.experimental.pallas import tpu_sc as plsc

N_BINS = 15
N_ROWS = 1_000_000
N_CLS = 64
NW = 32
CHUNK = 320
NCHUNK = 96
ROWS_W = CHUNK * NCHUNK
SC_ROWS = ROWS_W * NW
TC_ROWS = N_ROWS - SC_ROWS
GROUPS = CHUNK // 16


def _iota16():
    return lax.iota(jnp.int32, 16)


def _process_group(lbuf, labbuf, hists, g):
    cnt_h, acc_h, conf_h = hists
    lanes = _iota16()
    rows = g * 16 + lanes
    ms = [jnp.full((16,), -jnp.inf, dtype=jnp.float32) for _ in range(4)]
    ss = [jnp.zeros((16,), dtype=jnp.float32) for _ in range(4)]
    for j in range(N_CLS):
        v = plsc.load_gather(lbuf, [rows, (lanes + j) & (N_CLS - 1)])
        ss[j % 4] = ss[j % 4] + jnp.exp(v)
        ms[j % 4] = jnp.maximum(ms[j % 4], v)
    s = (ss[0] + ss[1]) + (ss[2] + ss[3])
    m = jnp.maximum(jnp.maximum(ms[0], ms[1]), jnp.maximum(ms[2], ms[3]))
    conf = jnp.exp(m) / s

    lab = plsc.load_gather(labbuf, [rows])
    v_lab = plsc.load_gather(lbuf, [rows, lab])
    acc = jnp.where(v_lab == m, 1.0, 0.0).astype(jnp.float32)

    binf = conf * jnp.float32(N_BINS)
    bi = binf.astype(jnp.int32)
    exact = (bi.astype(jnp.float32) == binf).astype(jnp.int32)
    bi = jnp.minimum(bi - exact, N_BINS - 1)

    slots = lanes * 16 + bi
    ones = jnp.ones((16,), dtype=jnp.float32)
    plsc.addupdate_scatter(cnt_h, [slots], ones)
    plsc.addupdate_scatter(acc_h, [slots], acc)
    plsc.addupdate_scatter(conf_h, [slots], conf)


def _sc_partials(logits, labels):
    mesh = plsc.VectorSubcoreMesh(core_axis_name="c", subcore_axis_name="s")

    @functools.partial(
        pl.kernel,
        mesh=mesh,
        out_type=jax.ShapeDtypeStruct((NW, 48), jnp.float32),
        compiler_params=pltpu.CompilerParams(needs_layout_passes=False),
        scratch_types=[
            pltpu.VMEM((CHUNK, N_CLS), jnp.float32),
            pltpu.VMEM((CHUNK, N_CLS), jnp.float32),
            pltpu.VMEM((CHUNK,), jnp.int32),
            pltpu.VMEM((CHUNK,), jnp.int32),
            pltpu.VMEM((256,), jnp.float32),
            pltpu.VMEM((256,), jnp.float32),
            pltpu.VMEM((256,), jnp.float32),
            pltpu.VMEM((48,), jnp.float32),
            pltpu.SemaphoreType.DMA,
            pltpu.SemaphoreType.DMA,
            pltpu.SemaphoreType.DMA,
            pltpu.SemaphoreType.DMA,
        ],
    )
    def body(logits_hbm, labels_hbm, out_hbm, lbuf0, lbuf1, labb0, labb1,
             cnt_h, acc_h, conf_h, stage, sg0, sg1, sb0, sb1):
        wid = lax.axis_index("s") * 2 + lax.axis_index("c")
        row0 = wid * ROWS_W
        lanes = _iota16()
        z16 = jnp.zeros((16,), dtype=jnp.float32)
        for k in range(16):
            cnt_h[pl.ds(k * 16, 16)] = z16
            acc_h[pl.ds(k * 16, 16)] = z16
            conf_h[pl.ds(k * 16, 16)] = z16
        hists = (cnt_h, acc_h, conf_h)

        def dma_logits(c, buf, sem):
            return pltpu.make_async_copy(
                logits_hbm.at[pl.ds(row0 + c * CHUNK, CHUNK)], buf, sem)

        def dma_labels(c, buf, sem):
            return pltpu.make_async_copy(
                labels_hbm.at[pl.ds(row0 + c * CHUNK, CHUNK)], buf, sem)

        def process(lbuf, labbuf):
            def g_body(g, carry):
                _process_group(lbuf, labbuf, hists, g)
                return carry
            lax.fori_loop(0, GROUPS, g_body, 0)

        dma_logits(0, lbuf0, sg0).start()
        dma_labels(0, labb0, sb0).start()

        def pair_body(c2, carry):
            c = c2 * 2
            dma_logits(c + 1, lbuf1, sg1).start()
            dma_labels(c + 1, labb1, sb1).start()
            dma_logits(c, lbuf0, sg0).wait()
            dma_labels(c, labb0, sb0).wait()
            process(lbuf0, labb0)

            @pl.when(c2 < (NCHUNK // 2) - 1)
            def _next():
                dma_logits(c + 2, lbuf0, sg0).start()
                dma_labels(c + 2, labb0, sb0).start()
            dma_logits(c + 1, lbuf1, sg1).wait()
            dma_labels(c + 1, labb1, sb1).wait()
            process(lbuf1, labb1)
            return carry
        lax.fori_loop(0, NCHUNK // 2, pair_body, 0)

        for h_idx, h in enumerate(hists):
            tot = z16
            for lane in range(16):
                tot = tot + plsc.load_gather(h, [lane * 16 + lanes])
            stage[pl.ds(h_idx * 16, 16)] = tot
        pltpu.sync_copy(stage, out_hbm.at[wid])

    return body(logits, labels)


def _combine_body(p_ref, tl_ref, tlab_ref, out_ref):
    l = tl_ref[...]
    lab = tlab_ref[...]
    b, c = l.shape
    m = jnp.max(l, axis=1, keepdims=True)
    z = jnp.sum(jnp.exp(l - m), axis=1, keepdims=True)
    conf = 1.0 / z
    iota = lax.broadcasted_iota(jnp.int32, (b, c), 1)
    pred = jnp.min(jnp.where(l == m, iota, c), axis=1, keepdims=True)
    acc = (pred == lab).astype(jnp.float32)

    bidx = lax.broadcasted_iota(jnp.int32, (1, N_BINS), 1).astype(jnp.float32)
    lows = bidx / N_BINS
    highs = (bidx + 1.0) / N_BINS
    in_bin = ((conf > lows) & (conf <= highs)).astype(jnp.float32)

    p = p_ref[...]
    cnt = (jnp.sum(p[:, 0:N_BINS], axis=0, keepdims=True)
           + jnp.sum(in_bin, axis=0, keepdims=True))
    asum = (jnp.sum(p[:, 16:16 + N_BINS], axis=0, keepdims=True)
            + jnp.sum(in_bin * acc, axis=0, keepdims=True))
    csum = (jnp.sum(p[:, 32:32 + N_BINS], axis=0, keepdims=True)
            + jnp.sum(in_bin * conf, axis=0, keepdims=True))
    prop = cnt / N_ROWS
    safe = jnp.maximum(cnt, 1.0)
    nonempty = (cnt > 0).astype(jnp.float32)
    per_bin = jnp.abs(csum / safe - asum / safe) * prop * nonempty
    out_ref[...] = jnp.sum(per_bin, axis=1, keepdims=True)


def kernel(logits, labels):
    labels = labels.astype(jnp.int32)
    partials = _sc_partials(logits, labels)
    tail_logits = logits[SC_ROWS:]
    tail_labels = labels[SC_ROWS:].reshape(TC_ROWS, 1)
    out = pl.pallas_call(
        _combine_body,
        out_shape=jax.ShapeDtypeStruct((1, 1), jnp.float32),
    )(partials, tail_logits, tail_labels)
    return out.reshape(1)
